# baseline (speedup 1.0000x reference)
import functools

import jax
import jax.numpy as jnp
from jax import lax
from jax.experimental import pallas as pl
from jax.experimental.pallas import tpu as pltpu
from jax.experimental.pallas import tpu_sc as plsc

T = 128
D = 1024
E = 64
K = 8
F = 512
FS = 2048
FS_CHUNK = 512
EPB = 2

SC_CORES = 2
SC_SUBCORES = 16
SC_LANES = 16
SC_WORKERS = SC_CORES * SC_SUBCORES
TOK_PER_W = T // SC_WORKERS
EV = E // SC_LANES


def _logits_body(x_ref, wr_ref, logits_ref):
    for w in range(T // SC_LANES):
        xw = x_ref[w * SC_LANES:(w + 1) * SC_LANES, :]
        logits_ref[w] = jax.lax.dot_general(
            wr_ref[:], xw, (((1,), (1,)), ((), ())),
            preferred_element_type=jnp.float32)


def _sc_router_body(logits_hbm, comb_hbm, lvm, ovm):
    wid = lax.axis_index("s") * SC_CORES + lax.axis_index("c")

    @pl.when(wid < T // SC_LANES)
    def _():
        pltpu.sync_copy(logits_hbm.at[wid], lvm)
        neg = jnp.full((SC_LANES,), -3.0e38, jnp.float32)
        top = [neg] * K
        for e in range(E):
            v = lvm[e, :]
            for j in range(K):
                new_tj = jnp.maximum(top[j], v)
                v = jnp.minimum(top[j], v)
                top[j] = new_tj
        mx = top[0]
        thresh = top[K - 1]
        ssum = jnp.zeros((SC_LANES,), jnp.float32)
        for j in range(K):
            ssum = ssum + jnp.exp(top[j] - mx)
        inv = jnp.float32(1.0) / ssum
        for e in range(E):
            le = lvm[e, :]
            ovm[e, :] = jnp.where(le >= thresh, jnp.exp(le - mx) * inv,
                                  jnp.float32(0.0))
        pltpu.sync_copy(ovm, comb_hbm.at[wid])


_sc_router = functools.partial(
    pl.kernel,
    out_type=jax.ShapeDtypeStruct((T // SC_LANES, E, SC_LANES), jnp.float32),
    mesh=plsc.VectorSubcoreMesh(core_axis_name="c", subcore_axis_name="s"),
    scratch_types=[
        pltpu.VMEM((E, SC_LANES), jnp.float32),
        pltpu.VMEM((E, SC_LANES), jnp.float32),
    ],
)(_sc_router_body)


NS = FS // FS_CHUNK


def _moe_body(x_ref, comb3_ref, wsg_ref, wsu_ref, wsd_ref, wseg_ref,
              wg_ref, wu_ref, wd_ref, out_ref, comb_ref):
    s = pl.program_id(0)
    x = x_ref[:]

    @pl.when(s == 0)
    def _():
        out_ref[:] = jnp.zeros_like(out_ref)
        for w in range(T // SC_LANES):
            comb_ref[w * SC_LANES:(w + 1) * SC_LANES, :] = comb3_ref[w].T

    @pl.when(s < NS)
    def _():
        g = jnp.dot(x, wsg_ref[:], preferred_element_type=jnp.float32)
        u = jnp.dot(x, wsu_ref[:], preferred_element_type=jnp.float32)
        h = (g * jax.nn.sigmoid(g)) * u
        out_ref[:] += jnp.dot(h, wsd_ref[:],
                              preferred_element_type=jnp.float32)

        @pl.when(s == NS - 1)
        def _():
            seg = jax.nn.sigmoid(jax.lax.dot_general(
                x, wseg_ref[:], (((1,), (1,)), ((), ())),
                preferred_element_type=jnp.float32))
            out_ref[:] = seg * out_ref[:]

    @pl.when(s >= NS)
    def _():
        b = s - NS
        lane = jax.lax.broadcasted_iota(jnp.int32, (T, E), 1)
        acc = jnp.zeros((T, D), jnp.float32)
        for j in range(EPB):
            g = jnp.dot(x, wg_ref[j], preferred_element_type=jnp.float32)
            u = jnp.dot(x, wu_ref[j], preferred_element_type=jnp.float32)
            h = (g * jax.nn.sigmoid(g)) * u
            y = jnp.dot(h, wd_ref[j], preferred_element_type=jnp.float32)
            scale = jnp.sum(jnp.where(lane == b * EPB + j, comb_ref[:], 0.0),
                            axis=1, keepdims=True)
            acc += scale * y
        out_ref[:] += acc


def kernel(hidden_states, w_router, w_gate, w_up, w_down,
           w_shared_gate_proj, w_shared_up_proj, w_shared_down_proj,
           w_shared_expert_gate):
    x = hidden_states.reshape(T, D)

    logits3 = pl.pallas_call(
        _logits_body,
        out_shape=jax.ShapeDtypeStruct((T // SC_LANES, E, SC_LANES),
                                       jnp.float32),
    )(x, w_router)
    comb3 = _sc_router(logits3)

    def _sidx(s):
        return jnp.minimum(s, NS - 1)

    def _eidx(s):
        return jnp.maximum(s - NS, 0)

    out = pl.pallas_call(
        _moe_body,
        grid=(NS + E // EPB,),
        in_specs=[
            pl.BlockSpec((T, D), lambda s: (0, 0)),
            pl.BlockSpec((T // SC_LANES, E, SC_LANES), lambda s: (0, 0, 0)),
            pl.BlockSpec((D, FS_CHUNK), lambda s: (0, _sidx(s))),
            pl.BlockSpec((D, FS_CHUNK), lambda s: (0, _sidx(s))),
            pl.BlockSpec((FS_CHUNK, D), lambda s: (_sidx(s), 0)),
            pl.BlockSpec((1, D), lambda s: (0, 0)),
            pl.BlockSpec((EPB, D, F), lambda s: (_eidx(s), 0, 0)),
            pl.BlockSpec((EPB, D, F), lambda s: (_eidx(s), 0, 0)),
            pl.BlockSpec((EPB, F, D), lambda s: (_eidx(s), 0, 0)),
        ],
        out_specs=pl.BlockSpec((T, D), lambda s: (0, 0)),
        out_shape=jax.ShapeDtypeStruct((T, D), jnp.float32),
        scratch_shapes=[pltpu.VMEM((T, E), jnp.float32)],
    )(x, comb3, w_shared_gate_proj, w_shared_up_proj, w_shared_down_proj,
      w_shared_expert_gate, w_gate, w_up, w_down)

    return out


# speedup vs baseline: 1.0200x; 1.0200x over previous
import functools

import jax
import jax.numpy as jnp
from jax import lax
from jax.experimental import pallas as pl
from jax.experimental.pallas import tpu as pltpu
from jax.experimental.pallas import tpu_sc as plsc

T = 128
D = 1024
E = 64
K = 8
F = 512
FS = 2048
FS_CHUNK = 512
EPB = 2

SC_CORES = 2
SC_LANES = 16


def _logits_body(x_ref, wr_ref, logits_ref):
    for w in range(T // SC_LANES):
        xw = x_ref[w * SC_LANES:(w + 1) * SC_LANES, :]
        logits_ref[w] = jax.lax.dot_general(
            wr_ref[:], xw, (((1,), (1,)), ((), ())),
            preferred_element_type=jnp.float32)


def _sc_router_body(logits_hbm, comb_hbm, lvm, ovm):
    wid = lax.axis_index("s") * SC_CORES + lax.axis_index("c")

    @pl.when(wid < T // SC_LANES)
    def _():
        pltpu.sync_copy(logits_hbm.at[wid], lvm)
        neg = jnp.full((SC_LANES,), -3.0e38, jnp.float32)
        top = [neg] * K
        for e in range(E):
            v = lvm[e, :]
            for j in range(K):
                new_tj = jnp.maximum(top[j], v)
                v = jnp.minimum(top[j], v)
                top[j] = new_tj
        mx = top[0]
        thresh = top[K - 1]
        ssum = jnp.zeros((SC_LANES,), jnp.float32)
        for j in range(K):
            ssum = ssum + jnp.exp(top[j] - mx)
        inv = jnp.float32(1.0) / ssum
        for e in range(E):
            le = lvm[e, :]
            ovm[e, :] = jnp.where(le >= thresh, jnp.exp(le - mx) * inv,
                                  jnp.float32(0.0))
        pltpu.sync_copy(ovm, comb_hbm.at[wid])


_sc_router = functools.partial(
    pl.kernel,
    out_type=jax.ShapeDtypeStruct((T // SC_LANES, E, SC_LANES), jnp.float32),
    mesh=plsc.VectorSubcoreMesh(core_axis_name="c", subcore_axis_name="s"),
    scratch_types=[
        pltpu.VMEM((E, SC_LANES), jnp.float32),
        pltpu.VMEM((E, SC_LANES), jnp.float32),
    ],
)(_sc_router_body)


def _shared_body(x_ref, wsg_ref, wsu_ref, wsd_ref, wseg_ref, out_ref):
    c = pl.program_id(0)
    x = x_ref[:]
    g = jnp.dot(x, wsg_ref[:], preferred_element_type=jnp.float32)
    u = jnp.dot(x, wsu_ref[:], preferred_element_type=jnp.float32)
    h = (g * jax.nn.sigmoid(g)) * u
    y = jnp.dot(h, wsd_ref[:], preferred_element_type=jnp.float32)

    @pl.when(c == 0)
    def _():
        out_ref[:] = jnp.zeros_like(out_ref)

    out_ref[:] += y

    @pl.when(c == (FS // FS_CHUNK) - 1)
    def _():
        seg = jax.nn.sigmoid(jax.lax.dot_general(
            x, wseg_ref[:], (((1,), (1,)), ((), ())),
            preferred_element_type=jnp.float32))
        out_ref[:] = seg * out_ref[:]


def _expert_body(x_ref, comb3_ref, shared_ref, wg_ref, wu_ref, wd_ref,
                 out_ref, comb_ref):
    b = pl.program_id(0)
    x = x_ref[:]
    lane = jax.lax.broadcasted_iota(jnp.int32, (T, E), 1)

    @pl.when(b == 0)
    def _():
        out_ref[:] = shared_ref[:]
        for w in range(T // SC_LANES):
            comb_ref[w * SC_LANES:(w + 1) * SC_LANES, :] = comb3_ref[w].T

    acc = jnp.zeros((T, D), jnp.float32)
    for j in range(EPB):
        g = jnp.dot(x, wg_ref[j], preferred_element_type=jnp.float32)
        u = jnp.dot(x, wu_ref[j], preferred_element_type=jnp.float32)
        h = (g * jax.nn.sigmoid(g)) * u
        y = jnp.dot(h, wd_ref[j], preferred_element_type=jnp.float32)
        scale = jnp.sum(jnp.where(lane == b * EPB + j, comb_ref[:], 0.0),
                        axis=1, keepdims=True)
        acc += scale * y
    out_ref[:] += acc


def kernel(hidden_states, w_router, w_gate, w_up, w_down,
           w_shared_gate_proj, w_shared_up_proj, w_shared_down_proj,
           w_shared_expert_gate):
    x = hidden_states.reshape(T, D)

    logits3 = pl.pallas_call(
        _logits_body,
        out_shape=jax.ShapeDtypeStruct((T // SC_LANES, E, SC_LANES),
                                       jnp.float32),
    )(x, w_router)
    comb3 = _sc_router(logits3)

    n_chunks = FS // FS_CHUNK
    shared_out = pl.pallas_call(
        _shared_body,
        grid=(n_chunks,),
        in_specs=[
            pl.BlockSpec((T, D), lambda c: (0, 0)),
            pl.BlockSpec((D, FS_CHUNK), lambda c: (0, c)),
            pl.BlockSpec((D, FS_CHUNK), lambda c: (0, c)),
            pl.BlockSpec((FS_CHUNK, D), lambda c: (c, 0)),
            pl.BlockSpec((1, D), lambda c: (0, 0)),
        ],
        out_specs=pl.BlockSpec((T, D), lambda c: (0, 0)),
        out_shape=jax.ShapeDtypeStruct((T, D), jnp.float32),
    )(x, w_shared_gate_proj, w_shared_up_proj, w_shared_down_proj,
      w_shared_expert_gate)

    out = pl.pallas_call(
        _expert_body,
        grid=(E // EPB,),
        in_specs=[
            pl.BlockSpec((T, D), lambda b: (0, 0)),
            pl.BlockSpec((T // SC_LANES, E, SC_LANES), lambda b: (0, 0, 0)),
            pl.BlockSpec((T, D), lambda b: (0, 0)),
            pl.BlockSpec((EPB, D, F), lambda b: (b, 0, 0)),
            pl.BlockSpec((EPB, D, F), lambda b: (b, 0, 0)),
            pl.BlockSpec((EPB, F, D), lambda b: (b, 0, 0)),
        ],
        out_specs=pl.BlockSpec((T, D), lambda b: (0, 0)),
        out_shape=jax.ShapeDtypeStruct((T, D), jnp.float32),
        scratch_shapes=[pltpu.VMEM((T, E), jnp.float32)],
    )(x, comb3, shared_out, w_gate, w_up, w_down)

    return out
